# baseline (speedup 1.0000x reference)
import math

import jax
import jax.numpy as jnp
import numpy as np
from jax.experimental import pallas as pl
from jax.experimental.pallas import tpu as pltpu

_N = 512
_V = 4
_F = 64
_H = 64
_BR = 16
_L = _BR * _N

_RS132 = 1.0 / math.sqrt(_V + 2 * _F)
_RS64 = 1.0 / math.sqrt(64.0)
_RS128 = 1.0 / math.sqrt(128.0)
_INV_DEG = 1.0 / (_N - 1)


def _sigmoid(x):
    return 0.5 * jnp.tanh(0.5 * x) + 0.5


def _silu(x):
    y = 0.5 * x
    return y * (jnp.tanh(y) + 1.0)


def _egcl_body(nv_ref, nf_ref, nvT_ref, nfT_ref, R_ref, RT_ref,
               We1vT_ref, We1sT_ref, We1rT_ref, We2T_ref,
               Wx1T_ref, Wx2T_ref, WinfT_ref, WlinT_ref, blin_ref,
               Wh1m_ref, Wh1f_ref, Wh2_ref, Wout_ref,
               vout_ref, fout_ref,
               nvT_t_ref, A_sT_t_ref):
    i = pl.program_id(0)
    r0 = i * _BR
    f32 = jnp.float32

    @pl.when(i == 0)
    def _():
        nvT = nvT_ref[:, :]
        A_sT = jnp.dot(We1sT_ref[:, :], nfT_ref[:, :],
                       preferred_element_type=f32)
        nvT_t_ref[:, :] = jnp.concatenate([nvT] * _BR, axis=1)
        A_sT_t_ref[:, :] = jnp.concatenate([A_sT] * _BR, axis=1)

    lane = jax.lax.broadcasted_iota(jnp.int32, (1, _L), 1)
    s_id = lane % _N
    r_id = r0 + lane // _N
    mask = (s_id != r_id).astype(f32)

    R = R_ref[:, :]
    RT = RT_ref[:, :]

    nf_blk = nf_ref[:, :]
    nv_blk = nv_ref[:, :]
    A_rb = jnp.dot(We1rT_ref[:, :], nf_blk.T,
                   preferred_element_type=f32)
    nv_r_exp = jnp.dot(nv_blk.T, R, preferred_element_type=f32)
    A_r_exp = jnp.dot(A_rb, R, preferred_element_type=f32)

    diff = nv_r_exp - nvT_t_ref[:, :]
    row = jax.lax.broadcasted_iota(jnp.int32, (_V, 12), 0)
    col = jax.lax.broadcasted_iota(jnp.int32, (_V, 12), 1)
    sel4 = (col // 3 == row).astype(f32)
    n2 = jnp.dot(sel4, diff * diff,
                 preferred_element_type=f32)
    length = jnp.sqrt(jnp.maximum(n2, 1e-20))

    bf16 = jnp.bfloat16
    h1 = jnp.dot(We1vT_ref[:, :], n2.astype(bf16),
                 preferred_element_type=f32)
    h1 = _silu(h1 + A_sT_t_ref[:, :] + A_r_exp)
    m = _silu(jnp.dot(We2T_ref[:, :], h1.astype(bf16),
                      preferred_element_type=f32))
    mb = m.astype(bf16)
    p = _silu(jnp.dot(Wx1T_ref[:, :], mb,
                      preferred_element_type=f32))
    p = _silu(jnp.dot(Wx2T_ref[:, :], p.astype(bf16),
                      preferred_element_type=f32))
    phi = jnp.dot(WlinT_ref[:, :], p.astype(bf16),
                  preferred_element_type=f32) + blin_ref[:, :]

    g = phi / (1.0 + length) * mask
    g12 = jnp.dot(sel4.T, g, preferred_element_type=f32)
    shifts = jnp.dot((g12 * diff).astype(bf16), RT,
                     preferred_element_type=f32)

    e = _sigmoid(jnp.dot(WinfT_ref[:, :], mb,
                         preferred_element_type=f32))
    m_i = jnp.dot((m * (e * mask)).astype(bf16), RT,
                  preferred_element_type=f32)

    m_i_n = m_i.T
    hh = jnp.dot(m_i_n, Wh1m_ref[:, :], preferred_element_type=f32)
    hh = hh + jnp.dot(nf_blk, Wh1f_ref[:, :], preferred_element_type=f32)
    hh = _silu(hh * _RS128)
    hh = _silu(jnp.dot(hh, Wh2_ref[:, :],
                       preferred_element_type=f32) * _RS64)
    fout_ref[:, :] = (jnp.dot(hh, Wout_ref[:, :],
                              preferred_element_type=f32) * _RS64 + nf_blk)

    vout_ref[:, :] = nv_blk + shifts.T * _INV_DEG


@jax.jit
def kernel(node_vectors, node_features, We1, We2, Wx1, Wx2, Winf, Wlin,
           blin, Wh1, Wh2, Wout):
    f32 = jnp.float32
    nv = node_vectors.reshape(_N, _V * 3).astype(f32)
    nvT = nv.T
    nf = node_features.astype(f32)
    nfT = nf.T

    seg = np.arange(_L) // _N
    R = jnp.asarray(seg[None, :] == np.arange(_BR)[:, None], dtype=f32)
    RT = jnp.asarray(seg[:, None] == np.arange(_BR)[None, :],
                     dtype=jnp.bfloat16)

    We1vT = We1[:_V].T * _RS132
    We1sT = We1[_V:_V + _F].T * _RS132
    We1rT = We1[_V + _F:].T * _RS132
    Wh1m = Wh1[:_H]
    Wh1f = Wh1[_H:]

    grid = (_N // _BR,)

    def full(shape):
        nd = len(shape)
        return pl.BlockSpec(shape, lambda i: (0,) * nd)

    out_shape = [
        jax.ShapeDtypeStruct((_N, 12), f32),
        jax.ShapeDtypeStruct((_N, _F), f32),
    ]
    out_specs = [
        pl.BlockSpec((_BR, 12), lambda i: (i, 0)),
        pl.BlockSpec((_BR, _F), lambda i: (i, 0)),
    ]
    in_arrays = [
        nv, nf, nvT, nfT, R, RT,
        We1vT.astype(jnp.bfloat16), We1sT, We1rT,
        (We2.T * _RS64).astype(jnp.bfloat16),
        (Wx1.T * _RS64).astype(jnp.bfloat16),
        (Wx2.T * _RS64).astype(jnp.bfloat16),
        (Winf.T * _RS64).astype(jnp.bfloat16),
        Wlin.T.astype(jnp.bfloat16),
        blin.reshape(_V, 1),
        Wh1m, Wh1f, Wh2, Wout,
    ]
    in_specs = [full(a.shape) for a in in_arrays]
    in_specs[0] = pl.BlockSpec((_BR, 12), lambda i: (i, 0))
    in_specs[1] = pl.BlockSpec((_BR, _F), lambda i: (i, 0))

    vout, fout = pl.pallas_call(
        _egcl_body,
        grid=grid,
        in_specs=in_specs,
        out_specs=out_specs,
        out_shape=out_shape,
        scratch_shapes=[
            pltpu.VMEM((12, _L), f32),
            pltpu.VMEM((64, _L), f32),
        ],
    )(*in_arrays)

    return vout.reshape(_N, _V, 3), fout


# speedup vs baseline: 1.0661x; 1.0661x over previous
import math

import jax
import jax.numpy as jnp
import numpy as np
from jax.experimental import pallas as pl
from jax.experimental.pallas import tpu as pltpu

_N = 512
_V = 4
_F = 64
_H = 64
_BR = 32
_L = _BR * _N

_RS132 = 1.0 / math.sqrt(_V + 2 * _F)
_RS64 = 1.0 / math.sqrt(64.0)
_RS128 = 1.0 / math.sqrt(128.0)
_INV_DEG = 1.0 / (_N - 1)


def _sigmoid(x):
    return 0.5 * jnp.tanh(0.5 * x) + 0.5


def _silu(x):
    y = 0.5 * x
    return y * (jnp.tanh(y) + 1.0)


def _egcl_body(nv_ref, nf_ref, nvT_ref, nfT_ref, R_ref, RT_ref,
               We1vT_ref, We1sT_ref, We1rT_ref, We2T_ref,
               Wx1T_ref, Wx2T_ref, WinfT_ref, WlinT_ref, blin_ref,
               Wh1m_ref, Wh1f_ref, Wh2_ref, Wout_ref,
               vout_ref, fout_ref,
               nvT_t_ref, A_sT_t_ref):
    i = pl.program_id(0)
    r0 = i * _BR
    f32 = jnp.float32

    @pl.when(i == 0)
    def _():
        nvT = nvT_ref[:, :]
        A_sT = jnp.dot(We1sT_ref[:, :], nfT_ref[:, :],
                       preferred_element_type=f32)
        nvT_t_ref[:, :] = jnp.concatenate([nvT] * _BR, axis=1)
        A_sT_t_ref[:, :] = jnp.concatenate([A_sT] * _BR, axis=1)

    lane = jax.lax.broadcasted_iota(jnp.int32, (1, _L), 1)
    s_id = lane % _N
    r_id = r0 + lane // _N
    mask = (s_id != r_id).astype(f32)

    R = R_ref[:, :]
    RT = RT_ref[:, :]

    nf_blk = nf_ref[:, :]
    nv_blk = nv_ref[:, :]
    A_rb = jnp.dot(We1rT_ref[:, :], nf_blk.T,
                   preferred_element_type=f32)
    nv_r_exp = jnp.dot(nv_blk.T, R, preferred_element_type=f32)
    A_r_exp = jnp.dot(A_rb, R, preferred_element_type=f32)

    diff = nv_r_exp - nvT_t_ref[:, :]
    row = jax.lax.broadcasted_iota(jnp.int32, (_V, 12), 0)
    col = jax.lax.broadcasted_iota(jnp.int32, (_V, 12), 1)
    sel4 = (col // 3 == row).astype(f32)
    n2 = jnp.dot(sel4, diff * diff,
                 preferred_element_type=f32)
    length = jnp.sqrt(jnp.maximum(n2, 1e-20))

    h1 = jnp.dot(We1vT_ref[:, :], n2, preferred_element_type=f32)
    h1 = _silu(h1 + A_sT_t_ref[:, :] + A_r_exp)
    m = _silu(jnp.dot(We2T_ref[:, :], h1,
                      preferred_element_type=f32))
    p = _silu(jnp.dot(Wx1T_ref[:, :], m,
                      preferred_element_type=f32))
    p = _silu(jnp.dot(Wx2T_ref[:, :], p,
                      preferred_element_type=f32))
    phi = jnp.dot(WlinT_ref[:, :], p,
                  preferred_element_type=f32) + blin_ref[:, :]

    g = phi / (1.0 + length) * mask
    g12 = jnp.dot(sel4.T, g, preferred_element_type=f32)
    shifts = jnp.dot(g12 * diff, RT,
                     preferred_element_type=f32)

    e = _sigmoid(jnp.dot(WinfT_ref[:, :], m,
                         preferred_element_type=f32))
    m_i = jnp.dot(m * (e * mask), RT,
                  preferred_element_type=f32)

    m_i_n = m_i.T
    hh = jnp.dot(m_i_n, Wh1m_ref[:, :], preferred_element_type=f32)
    hh = hh + jnp.dot(nf_blk, Wh1f_ref[:, :], preferred_element_type=f32)
    hh = _silu(hh * _RS128)
    hh = _silu(jnp.dot(hh, Wh2_ref[:, :],
                       preferred_element_type=f32) * _RS64)
    fout_ref[:, :] = (jnp.dot(hh, Wout_ref[:, :],
                              preferred_element_type=f32) * _RS64 + nf_blk)

    vout_ref[:, :] = nv_blk + shifts.T * _INV_DEG


@jax.jit
def kernel(node_vectors, node_features, We1, We2, Wx1, Wx2, Winf, Wlin,
           blin, Wh1, Wh2, Wout):
    f32 = jnp.float32
    nv = node_vectors.reshape(_N, _V * 3).astype(f32)
    nvT = nv.T
    nf = node_features.astype(f32)
    nfT = nf.T

    seg = np.arange(_L) // _N
    R = jnp.asarray(seg[None, :] == np.arange(_BR)[:, None], dtype=f32)
    RT = jnp.asarray(seg[:, None] == np.arange(_BR)[None, :], dtype=f32)

    We1vT = We1[:_V].T * _RS132
    We1sT = We1[_V:_V + _F].T * _RS132
    We1rT = We1[_V + _F:].T * _RS132
    Wh1m = Wh1[:_H]
    Wh1f = Wh1[_H:]

    grid = (_N // _BR,)

    def full(shape):
        nd = len(shape)
        return pl.BlockSpec(shape, lambda i: (0,) * nd)

    out_shape = [
        jax.ShapeDtypeStruct((_N, 12), f32),
        jax.ShapeDtypeStruct((_N, _F), f32),
    ]
    out_specs = [
        pl.BlockSpec((_BR, 12), lambda i: (i, 0)),
        pl.BlockSpec((_BR, _F), lambda i: (i, 0)),
    ]
    in_arrays = [
        nv, nf, nvT, nfT, R, RT,
        We1vT, We1sT, We1rT, We2.T * _RS64,
        Wx1.T * _RS64, Wx2.T * _RS64, Winf.T * _RS64, Wlin.T,
        blin.reshape(_V, 1),
        Wh1m, Wh1f, Wh2, Wout,
    ]
    in_specs = [full(a.shape) for a in in_arrays]
    in_specs[0] = pl.BlockSpec((_BR, 12), lambda i: (i, 0))
    in_specs[1] = pl.BlockSpec((_BR, _F), lambda i: (i, 0))

    vout, fout = pl.pallas_call(
        _egcl_body,
        grid=grid,
        in_specs=in_specs,
        out_specs=out_specs,
        out_shape=out_shape,
        scratch_shapes=[
            pltpu.VMEM((12, _L), f32),
            pltpu.VMEM((64, _L), f32),
        ],
    )(*in_arrays)

    return vout.reshape(_N, _V, 3), fout
